# baseline (speedup 1.0000x reference)
import functools

import jax
import jax.numpy as jnp
from jax import lax
from jax.experimental import pallas as pl
from jax.experimental.pallas import tpu as pltpu
from jax.experimental.pallas import tpu_sc as plsc

N = 10000
E = 320000
F = 128
PN = 10240
NC = 2
NS = 16
CH = 80
NCHUNK = 125
G = 5
NGROUP = NCHUNK // G
G5 = 25
NCH3 = 132
PT = NCH3 * CH
E3 = NC * NS * PT

_MESH = plsc.VectorSubcoreMesh(
    core_axis_name="c", subcore_axis_name="s", num_cores=NC, num_subcores=NS
)


def _fill(ref, n, value):
    vec = jnp.full((16,), value, jnp.float32)

    def body(i, _):
        ref[pl.ds(i * 16, 16)] = vec
        return 0

    lax.fori_loop(0, n // 16, body, 0)


@functools.partial(
    pl.kernel,
    out_type=jax.ShapeDtypeStruct((PN,), jnp.float32),
    mesh=_MESH,
    scratch_types=[
        pltpu.VMEM((2 * NCHUNK, CH), jnp.int32),
        pltpu.VMEM((CH,), jnp.float32),
        pltpu.VMEM((640,), jnp.float32),
        pltpu.SemaphoreType.DMA,
        pltpu.VMEM_SHARED((PN,), jnp.float32),
    ],
)
def _k1(dst3_hbm, hist_hbm, idx_v, ones_v, zbuf, sem, acc):
    cid = lax.axis_index("c")
    sid = lax.axis_index("s")
    _fill(ones_v, CH, 1.0)
    _fill(zbuf, 640, 0.0)
    pltpu.sync_copy(zbuf, acc.at[pl.ds(sid * 640, 640)])
    pltpu.sync_copy(dst3_hbm.at[2 * sid], idx_v.at[pl.ds(0, NCHUNK)])
    pltpu.sync_copy(dst3_hbm.at[2 * sid + 1], idx_v.at[pl.ds(NCHUNK, NCHUNK)])
    plsc.subcore_barrier()

    def body(c, _):
        pltpu.async_copy(ones_v, acc.at[idx_v.at[c]], sem, add=True)
        return 0

    lax.fori_loop(0, 2 * NCHUNK, body, 0)

    def drain(c, _):
        pltpu.make_async_copy(ones_v, acc.at[idx_v.at[c]], sem).wait()
        return 0

    lax.fori_loop(0, 2 * NCHUNK, drain, 0)
    plsc.subcore_barrier()

    @pl.when(cid == 0)
    def _():
        pltpu.sync_copy(acc.at[pl.ds(sid * 640, 640)],
                        hist_hbm.at[pl.ds(sid * 640, 640)])


def _k2_body(hist_ref, x_ref, w1_ref, y_ref, dinv_ref):
    dinv = lax.rsqrt(hist_ref[...] + 1.0)
    xw = jnp.dot(x_ref[...], w1_ref[...], preferred_element_type=jnp.float32)
    y_ref[pl.ds(0, N), :] = xw * dinv[:N, None]
    y_ref[pl.ds(N, PN - N), :] = jnp.zeros((PN - N, F), jnp.float32)
    dinv_ref[...] = dinv


def _k2(hist, x, W1):
    return pl.pallas_call(
        _k2_body,
        out_shape=(
            jax.ShapeDtypeStruct((PN, F), jnp.float32),
            jax.ShapeDtypeStruct((PN,), jnp.float32),
        ),
    )(hist, x, W1)


@functools.partial(
    pl.kernel,
    out_type=jax.ShapeDtypeStruct((NC, PN, F), jnp.float32),
    mesh=_MESH,
    scratch_types=[
        pltpu.VMEM((PT,), jnp.int32),
        pltpu.VMEM((6, CH), jnp.int32),
        pltpu.VMEM((CH, F), jnp.float32),
        pltpu.VMEM((CH, F), jnp.float32),
        pltpu.VMEM((CH, F), jnp.float32),
        pltpu.SemaphoreType.DMA,
        pltpu.SemaphoreType.DMA,
        pltpu.SemaphoreType.DMA,
        pltpu.SemaphoreType.DMA,
        pltpu.SemaphoreType.DMA,
        pltpu.SemaphoreType.DMA,
        pltpu.SemaphoreType.DMA,
        pltpu.SemaphoreType.DMA,
        pltpu.SemaphoreType.DMA,
        pltpu.VMEM_SHARED((PN, F), jnp.float32),
    ],
)
def _k3(src1_hbm, dst3_hbm, y_hbm, out_hbm, idxs, idxd, bufa, bufb, bufc,
        gsa, gsb, gsc, is0, is1, is2, is3, is4, is5, acc):
    cid = lax.axis_index("c")
    sid = lax.axis_index("s")
    w = cid * NS + sid
    bufs = (bufa, bufb, bufc)
    gsems = (gsa, gsb, gsc)
    isems = (is0, is1, is2, is3, is4, is5)

    def zrow(i, _):
        for k in range(F // 16):
            bufa[i, pl.ds(16 * k, 16)] = jnp.zeros((16,), jnp.float32)
        return 0

    lax.fori_loop(0, CH, zrow, 0)
    for q in range(8):
        pltpu.sync_copy(bufa, acc.at[pl.ds(sid * 640 + q * CH, CH)])
    pltpu.sync_copy(src1_hbm.at[pl.ds(w * PT, PT)], idxs)
    pltpu.sync_copy(dst3_hbm.at[w].at[pl.ds(0, 6)], idxd)
    plsc.subcore_barrier()

    def gidx(c):
        return idxs.at[pl.ds(c * CH, CH)]

    def fire_gather(c, k):
        pltpu.async_copy(y_hbm.at[gidx(c)], bufs[k], gsems[k])

    def wait_gather(c, k):
        pltpu.make_async_copy(y_hbm.at[gidx(c)], bufs[k], gsems[k]).wait()

    def fire_refill(c, s):
        pltpu.async_copy(dst3_hbm.at[w].at[c], idxd.at[lax.rem(c, 6)],
                         isems[s])

    def wait_refill(s):
        pltpu.make_async_copy(dst3_hbm.at[w].at[0], idxd.at[0], isems[s]).wait()

    def scatter(c, k):
        pltpu.sync_copy(bufs[k], acc.at[idxd.at[lax.rem(c, 6)]], add=True)

    for c in range(3):
        fire_gather(c, c)
    for c in range(6):
        k = c % 3
        wait_gather(c, k)
        scatter(c, k)
        fire_refill(c + 6, c % 6)
        fire_gather(c + 3, k)

    def body(g, _):
        for j in range(6):
            c = 6 + 6 * g + j
            k = j % 3
            wait_gather(c, k)
            wait_refill(j)
            scatter(c, k)
            fire_refill(c + 6, j)
            fire_gather(c + 3, k)
        return 0

    lax.fori_loop(0, (NCH3 - 12) // 6, body, 0)
    for c in range(NCH3 - 6, NCH3):
        k = c % 3
        wait_gather(c, k)
        wait_refill(c % 6)
        scatter(c, k)
        if c + 3 < NCH3:
            fire_gather(c + 3, k)
    plsc.subcore_barrier()
    pltpu.sync_copy(acc.at[pl.ds(sid * 640, 640)],
                    out_hbm.at[cid, pl.ds(sid * 640, 640)])


def _k4_body(p_ref, y_ref, dinv_ref, b1_ref, w2_ref, b2_ref, zs_ref, u_ref):
    dinv = dinv_ref[...]
    agg = p_ref[0] + p_ref[1] + y_ref[...]
    h = jax.nn.relu(agg * dinv[:, None] + b1_ref[...][None, :])
    z = jnp.dot(h, w2_ref[...], preferred_element_type=jnp.float32)[:, 0]
    zs = z * dinv
    zs_ref[...] = zs
    u_ref[...] = dinv * zs + b2_ref[0]


def _k4(partials, y, dinv, b1, W2, b2):
    return pl.pallas_call(
        _k4_body,
        out_shape=(
            jax.ShapeDtypeStruct((PN,), jnp.float32),
            jax.ShapeDtypeStruct((PN,), jnp.float32),
        ),
    )(partials, y, dinv, b1, W2, b2)


@functools.partial(
    pl.kernel,
    out_type=jax.ShapeDtypeStruct((NC, PN), jnp.float32),
    mesh=_MESH,
    scratch_types=[
        pltpu.VMEM((NCHUNK, CH), jnp.int32),
        pltpu.VMEM((NCHUNK, CH), jnp.int32),
        pltpu.VMEM((G5 * CH,), jnp.float32),
        pltpu.SemaphoreType.DMA,
        pltpu.SemaphoreType.DMA,
        pltpu.VMEM((640,), jnp.float32),
        pltpu.VMEM_SHARED((PN,), jnp.float32),
        pltpu.VMEM_SHARED((PN,), jnp.float32),
    ],
)
def _k5(src3_hbm, dst3_hbm, zs_hbm, out_hbm,
        idxs, idxd, vals, gsem, ssem, zbuf, acc, ztab):
    cid = lax.axis_index("c")
    sid = lax.axis_index("s")
    w = cid * NS + sid
    _fill(zbuf, 640, 0.0)
    pltpu.sync_copy(zbuf, acc.at[pl.ds(sid * 640, 640)])
    pltpu.sync_copy(zs_hbm.at[pl.ds(sid * 640, 640)],
                    ztab.at[pl.ds(sid * 640, 640)])
    pltpu.sync_copy(src3_hbm.at[w], idxs)
    pltpu.sync_copy(dst3_hbm.at[w], idxd)
    plsc.subcore_barrier()

    def group(g, _):
        gds = []
        for j in range(G5):
            gds.append(pltpu.async_copy(
                ztab.at[idxs.at[g * G5 + j]],
                vals.at[pl.ds(j * CH, CH)], gsem))
        for d in gds:
            d.wait()
        sds = []
        for j in range(G5):
            sds.append(pltpu.async_copy(
                vals.at[pl.ds(j * CH, CH)],
                acc.at[idxd.at[g * G5 + j]], ssem, add=True))
        for d in sds:
            d.wait()
        return 0

    lax.fori_loop(0, NCHUNK // G5, group, 0)
    plsc.subcore_barrier()
    pltpu.sync_copy(acc.at[pl.ds(sid * 640, 640)],
                    out_hbm.at[cid, pl.ds(sid * 640, 640)])


def _k6_body(p_ref, dinv_ref, u_ref, o_ref):
    o_ref[...] = dinv_ref[...] * (p_ref[0] + p_ref[1]) + u_ref[...]


def _k6(partials2, dinv, u):
    return pl.pallas_call(
        _k6_body,
        out_shape=jax.ShapeDtypeStruct((PN,), jnp.float32),
    )(partials2, dinv, u)


def kernel(x, edge_index, W1, b1, W2, b2):
    src3 = edge_index[0].reshape(NC * NS, NCHUNK, CH)
    dst3 = edge_index[1].reshape(NC * NS, NCHUNK, CH)
    npad = E3 - E
    srcp = jnp.concatenate([edge_index[0], jnp.zeros((npad,), jnp.int32)])
    dstp = jnp.concatenate(
        [edge_index[1],
         N + (jnp.arange(npad, dtype=jnp.int32) % (PN - N))])
    dst3p = dstp.reshape(NC * NS, NCH3, CH)
    hist = _k1(dst3)
    y, dinv = _k2(hist, x, W1)
    partials = _k3(srcp, dst3p, y)
    zs, u = _k4(partials, y, dinv, b1, W2, b2)
    partials2 = _k5(src3, dst3, zs)
    out = _k6(partials2, dinv, u)
    return out[:N]


# speedup vs baseline: 7.2376x; 7.2376x over previous
import functools

import jax
import jax.numpy as jnp
from jax import lax
from jax.experimental import pallas as pl
from jax.experimental.pallas import tpu as pltpu
from jax.experimental.pallas import tpu_sc as plsc

N = 10000
E = 320000
F = 128
PN = 10240
NC = 2
NS = 16
CH = 80
NCHUNK = 125
G = 5
NGROUP = NCHUNK // G
G5 = 25
NCH3 = 132
PT = NCH3 * CH
E3 = NC * NS * PT

_MESH = plsc.VectorSubcoreMesh(
    core_axis_name="c", subcore_axis_name="s", num_cores=NC, num_subcores=NS
)


def _fill(ref, n, value):
    vec = jnp.full((16,), value, jnp.float32)

    def body(i, _):
        ref[pl.ds(i * 16, 16)] = vec
        return 0

    lax.fori_loop(0, n // 16, body, 0)


@functools.partial(
    pl.kernel,
    out_type=jax.ShapeDtypeStruct((PN,), jnp.float32),
    mesh=_MESH,
    scratch_types=[
        pltpu.VMEM((2 * NCHUNK, CH), jnp.int32),
        pltpu.VMEM((CH,), jnp.float32),
        pltpu.VMEM((640,), jnp.float32),
        pltpu.SemaphoreType.DMA,
        pltpu.VMEM_SHARED((PN,), jnp.float32),
    ],
)
def _k1(dst3_hbm, hist_hbm, idx_v, ones_v, zbuf, sem, acc):
    cid = lax.axis_index("c")
    sid = lax.axis_index("s")
    _fill(ones_v, CH, 1.0)
    _fill(zbuf, 640, 0.0)
    pltpu.sync_copy(zbuf, acc.at[pl.ds(sid * 640, 640)])
    pltpu.sync_copy(dst3_hbm.at[2 * sid], idx_v.at[pl.ds(0, NCHUNK)])
    pltpu.sync_copy(dst3_hbm.at[2 * sid + 1], idx_v.at[pl.ds(NCHUNK, NCHUNK)])
    plsc.subcore_barrier()

    def body(c, _):
        pltpu.async_copy(ones_v, acc.at[idx_v.at[c]], sem, add=True)
        return 0

    lax.fori_loop(0, 2 * NCHUNK, body, 0)

    def drain(c, _):
        pltpu.make_async_copy(ones_v, acc.at[idx_v.at[c]], sem).wait()
        return 0

    lax.fori_loop(0, 2 * NCHUNK, drain, 0)
    plsc.subcore_barrier()

    @pl.when(cid == 0)
    def _():
        pltpu.sync_copy(acc.at[pl.ds(sid * 640, 640)],
                        hist_hbm.at[pl.ds(sid * 640, 640)])


def _k2_body(hist_ref, x_ref, w1_ref, y_ref, dinv_ref):
    dinv = lax.rsqrt(hist_ref[...] + 1.0)
    xw = jnp.dot(x_ref[...], w1_ref[...], preferred_element_type=jnp.float32)
    y_ref[pl.ds(0, N), :] = xw * dinv[:N, None]
    y_ref[pl.ds(N, PN - N), :] = jnp.zeros((PN - N, F), jnp.float32)
    dinv_ref[...] = dinv


def _k2(hist, x, W1):
    return pl.pallas_call(
        _k2_body,
        out_shape=(
            jax.ShapeDtypeStruct((PN, F), jnp.float32),
            jax.ShapeDtypeStruct((PN,), jnp.float32),
        ),
    )(hist, x, W1)


@functools.partial(
    pl.kernel,
    out_type=jax.ShapeDtypeStruct((NC, PN, F), jnp.float32),
    mesh=_MESH,
    scratch_types=[
        pltpu.VMEM((PT,), jnp.int32),
        pltpu.VMEM((6, CH), jnp.int32),
        pltpu.VMEM((CH, F), jnp.float32),
        pltpu.VMEM((CH, F), jnp.float32),
        pltpu.VMEM((CH, F), jnp.float32),
        pltpu.SemaphoreType.DMA,
        pltpu.SemaphoreType.DMA,
        pltpu.SemaphoreType.DMA,
        pltpu.SemaphoreType.DMA,
        pltpu.SemaphoreType.DMA,
        pltpu.SemaphoreType.DMA,
        pltpu.SemaphoreType.DMA,
        pltpu.SemaphoreType.DMA,
        pltpu.SemaphoreType.DMA,
        pltpu.VMEM_SHARED((PN, F), jnp.float32),
    ],
)
def _k3(src1_hbm, dst3_hbm, y_hbm, out_hbm, idxs, idxd, bufa, bufb, bufc,
        gsa, gsb, gsc, is0, is1, is2, is3, is4, is5, acc):
    cid = lax.axis_index("c")
    sid = lax.axis_index("s")
    w = cid * NS + sid
    bufs = (bufa, bufb, bufc)
    gsems = (gsa, gsb, gsc)
    isems = (is0, is1, is2, is3, is4, is5)

    def zrow(i, _):
        for k in range(F // 16):
            bufa[i, pl.ds(16 * k, 16)] = jnp.zeros((16,), jnp.float32)
        return 0

    lax.fori_loop(0, CH, zrow, 0)
    for q in range(8):
        pltpu.sync_copy(bufa, acc.at[pl.ds(sid * 640 + q * CH, CH)])
    pltpu.sync_copy(src1_hbm.at[pl.ds(w * PT, PT)], idxs)
    pltpu.sync_copy(dst3_hbm.at[w].at[pl.ds(0, 6)], idxd)
    plsc.subcore_barrier()

    def gidx(c):
        return idxs.at[pl.ds(c * CH, CH)]

    def fire_gather(c, k):
        pltpu.async_copy(y_hbm.at[gidx(c)], bufs[k], gsems[k])

    def wait_gather(c, k):
        pltpu.make_async_copy(y_hbm.at[gidx(c)], bufs[k], gsems[k]).wait()

    def fire_refill(c, s):
        pltpu.async_copy(dst3_hbm.at[w].at[c], idxd.at[lax.rem(c, 6)],
                         isems[s])

    def wait_refill(s):
        pltpu.make_async_copy(dst3_hbm.at[w].at[0], idxd.at[0], isems[s]).wait()

    def scatter(c, k):
        pltpu.sync_copy(bufs[k], acc.at[idxd.at[lax.rem(c, 6)]], add=True)

    for c in range(3):
        fire_gather(c, c)
    for c in range(6):
        k = c % 3
        wait_gather(c, k)
        scatter(c, k)
        fire_refill(c + 6, c % 6)
        fire_gather(c + 3, k)

    def body(g, _):
        for j in range(6):
            c = 6 + 6 * g + j
            k = j % 3
            wait_gather(c, k)
            wait_refill(j)
            scatter(c, k)
            fire_refill(c + 6, j)
            fire_gather(c + 3, k)
        return 0

    lax.fori_loop(0, (NCH3 - 12) // 6, body, 0)
    for c in range(NCH3 - 6, NCH3):
        k = c % 3
        wait_gather(c, k)
        wait_refill(c % 6)
        scatter(c, k)
        if c + 3 < NCH3:
            fire_gather(c + 3, k)
    plsc.subcore_barrier()
    pltpu.sync_copy(acc.at[pl.ds(sid * 640, 640)],
                    out_hbm.at[cid, pl.ds(sid * 640, 640)])


def _k4_body(p_ref, y_ref, dinv_ref, b1_ref, w2_ref, b2_ref, zs_ref, u_ref):
    dinv = dinv_ref[...]
    agg = p_ref[0] + p_ref[1] + y_ref[...]
    h = jax.nn.relu(agg * dinv[:, None] + b1_ref[...][None, :])
    z = jnp.dot(h, w2_ref[...], preferred_element_type=jnp.float32)[:, 0]
    zs = z * dinv
    zs_ref[...] = zs
    u_ref[...] = dinv * zs + b2_ref[0]


def _k4(partials, y, dinv, b1, W2, b2):
    return pl.pallas_call(
        _k4_body,
        out_shape=(
            jax.ShapeDtypeStruct((PN,), jnp.float32),
            jax.ShapeDtypeStruct((PN,), jnp.float32),
        ),
    )(partials, y, dinv, b1, W2, b2)


@functools.partial(
    pl.kernel,
    out_type=jax.ShapeDtypeStruct((NC, PN), jnp.float32),
    mesh=_MESH,
    scratch_types=[
        pltpu.VMEM((NCHUNK, CH), jnp.int32),
        pltpu.VMEM((NCHUNK, CH), jnp.int32),
        pltpu.VMEM((G5 * CH,), jnp.float32),
        pltpu.SemaphoreType.DMA,
        pltpu.SemaphoreType.DMA,
        pltpu.VMEM((640,), jnp.float32),
        pltpu.VMEM_SHARED((PN,), jnp.float32),
        pltpu.VMEM_SHARED((PN,), jnp.float32),
    ],
)
def _k5(src3_hbm, dst3_hbm, zs_hbm, out_hbm,
        idxs, idxd, vals, gsem, ssem, zbuf, acc, ztab):
    cid = lax.axis_index("c")
    sid = lax.axis_index("s")
    w = cid * NS + sid
    _fill(zbuf, 640, 0.0)
    pltpu.sync_copy(zbuf, acc.at[pl.ds(sid * 640, 640)])
    pltpu.sync_copy(zs_hbm.at[pl.ds(sid * 640, 640)],
                    ztab.at[pl.ds(sid * 640, 640)])
    pltpu.sync_copy(src3_hbm.at[w], idxs)
    pltpu.sync_copy(dst3_hbm.at[w], idxd)
    plsc.subcore_barrier()

    def group(g, _):
        gds = []
        for j in range(G5):
            gds.append(pltpu.async_copy(
                ztab.at[idxs.at[g * G5 + j]],
                vals.at[pl.ds(j * CH, CH)], gsem))
        for d in gds:
            d.wait()
        sds = []
        for j in range(G5):
            sds.append(pltpu.async_copy(
                vals.at[pl.ds(j * CH, CH)],
                acc.at[idxd.at[g * G5 + j]], ssem, add=True))
        for d in sds:
            d.wait()
        return 0

    lax.fori_loop(0, NCHUNK // G5, group, 0)
    plsc.subcore_barrier()
    pltpu.sync_copy(acc.at[pl.ds(sid * 640, 640)],
                    out_hbm.at[cid, pl.ds(sid * 640, 640)])


def _k6_body(p_ref, dinv_ref, u_ref, o_ref):
    o_ref[...] = dinv_ref[...] * (p_ref[0] + p_ref[1]) + u_ref[...]


def _k6(partials2, dinv, u):
    return pl.pallas_call(
        _k6_body,
        out_shape=jax.ShapeDtypeStruct((PN,), jnp.float32),
    )(partials2, dinv, u)


def kernel(x, edge_index, W1, b1, W2, b2):
    src3 = edge_index[0].reshape(NC * NS, NCHUNK, CH)
    dst3 = edge_index[1].reshape(NC * NS, NCHUNK, CH)
    npad = E3 - E
    srcp = jnp.concatenate(
        [edge_index[0], jnp.arange(npad, dtype=jnp.int32) % N])
    dstp = jnp.concatenate(
        [edge_index[1],
         N + (jnp.arange(npad, dtype=jnp.int32) % (PN - N))])
    dst3p = dstp.reshape(NC * NS, NCH3, CH)
    hist = _k1(dst3)
    y, dinv = _k2(hist, x, W1)
    partials = _k3(srcp, dst3p, y)
    zs, u = _k4(partials, y, dinv, b1, W2, b2)
    partials2 = _k5(src3, dst3, zs)
    out = _k6(partials2, dinv, u)
    return out[:N]
